# baseline (speedup 1.0000x reference)
import functools

import jax
import jax.numpy as jnp
from jax.experimental import pallas as pl
from jax.experimental.pallas import tpu as pltpu

BLK = 128


def _fused_body(q_ref, k_ref, v_ref, w_ref, o_ref, *, heads, dh, scale):
    d = heads * dh
    acc = jnp.zeros((BLK, d), dtype=jnp.float32)
    for h in range(heads):
        qh = q_ref[0, h].astype(jnp.bfloat16)
        kh = k_ref[0, h].astype(jnp.bfloat16)
        vh = v_ref[0, h].astype(jnp.bfloat16)
        s = jax.lax.dot_general(
            qh, kh, (((1,), (1,)), ((), ())),
            preferred_element_type=jnp.float32) * scale
        m = jnp.max(s, axis=-1, keepdims=True)
        e = jnp.exp(s - m)
        p = (e / jnp.sum(e, axis=-1, keepdims=True)).astype(jnp.bfloat16)
        oh = jax.lax.dot_general(
            p, vh, (((1,), (0,)), ((), ())),
            preferred_element_type=jnp.float32)
        acc = acc + jax.lax.dot_general(
            oh.astype(jnp.bfloat16), w_ref[h * dh:(h + 1) * dh, :],
            (((1,), (0,)), ((), ())),
            preferred_element_type=jnp.float32)
    o_ref[0] = acc


def kernel(q, k, v, W_o):
    B, H, S, dh = q.shape
    D = H * dh
    nb = S // BLK
    scale = 1.0 / (dh ** 0.5)
    body = functools.partial(_fused_body, heads=H, dh=dh, scale=scale)
    qkv_spec = pl.BlockSpec((1, H, BLK, dh), lambda b, n: (b, 0, n, 0))
    out = pl.pallas_call(
        body,
        grid=(B, nb),
        in_specs=[
            qkv_spec,
            qkv_spec,
            qkv_spec,
            pl.BlockSpec((D, D), lambda b, n: (0, 0)),
        ],
        out_specs=pl.BlockSpec((1, BLK, D), lambda b, n: (b, n, 0)),
        out_shape=jax.ShapeDtypeStruct((B, S, D), jnp.float32),
        compiler_params=pltpu.CompilerParams(
            dimension_semantics=("parallel", "arbitrary"),
        ),
    )(q, k, v, W_o.astype(jnp.bfloat16))
    return out


# speedup vs baseline: 2.3984x; 2.3984x over previous
import functools

import jax
import jax.numpy as jnp
from jax.experimental import pallas as pl
from jax.experimental.pallas import tpu as pltpu

BLK = 128


def _fused_body(q_ref, k_ref, v_ref, w_ref, o_ref, *, heads, dh, scale):
    qa = q_ref[0].astype(jnp.bfloat16)
    ka = k_ref[0].astype(jnp.bfloat16)
    va = v_ref[0].astype(jnp.bfloat16)
    s = jax.lax.dot_general(
        qa, ka, (((2,), (2,)), ((0,), (0,))),
        preferred_element_type=jnp.float32) * scale
    m = jnp.max(s, axis=-1, keepdims=True)
    e = jnp.exp(s - m)
    p = (e / jnp.sum(e, axis=-1, keepdims=True)).astype(jnp.bfloat16)
    o = jax.lax.dot_general(
        p, va, (((2,), (1,)), ((0,), (0,))),
        preferred_element_type=jnp.float32)
    ob = o.astype(jnp.bfloat16)
    wa = w_ref[...].reshape(heads, dh, heads * dh)
    proj = jax.lax.dot_general(
        ob, wa, (((2,), (1,)), ((0,), (0,))),
        preferred_element_type=jnp.float32)
    o_ref[0] = jnp.sum(proj, axis=0)


def kernel(q, k, v, W_o):
    B, H, S, dh = q.shape
    D = H * dh
    nb = S // BLK
    scale = 1.0 / (dh ** 0.5)
    body = functools.partial(_fused_body, heads=H, dh=dh, scale=scale)
    qkv_spec = pl.BlockSpec((1, H, BLK, dh), lambda b, n: (b, 0, n, 0))
    out = pl.pallas_call(
        body,
        grid=(B, nb),
        in_specs=[
            qkv_spec,
            qkv_spec,
            qkv_spec,
            pl.BlockSpec((D, D), lambda b, n: (0, 0)),
        ],
        out_specs=pl.BlockSpec((1, BLK, D), lambda b, n: (b, n, 0)),
        out_shape=jax.ShapeDtypeStruct((B, S, D), jnp.float32),
        compiler_params=pltpu.CompilerParams(
            dimension_semantics=("parallel", "parallel"),
        ),
    )(q, k, v, W_o.astype(jnp.bfloat16))
    return out


# speedup vs baseline: 2.8768x; 1.1995x over previous
import functools

import jax
import jax.numpy as jnp
from jax.experimental import pallas as pl
from jax.experimental.pallas import tpu as pltpu

BLK = 128
NBLK = 2


def _fused_body(q_ref, k_ref, v_ref, w_ref, o_ref, *, heads, dh, scale):
    rows = NBLK * BLK
    hb = heads * NBLK
    qa = q_ref[0].astype(jnp.bfloat16).reshape(hb, BLK, dh)
    ka = k_ref[0].astype(jnp.bfloat16).reshape(hb, BLK, dh)
    va = v_ref[0].astype(jnp.bfloat16).reshape(hb, BLK, dh)
    s = jax.lax.dot_general(
        qa, ka, (((2,), (2,)), ((0,), (0,))),
        preferred_element_type=jnp.float32) * scale
    m = jnp.max(s, axis=-1, keepdims=True)
    e = jnp.exp(s - m)
    p = (e / jnp.sum(e, axis=-1, keepdims=True)).astype(jnp.bfloat16)
    o = jax.lax.dot_general(
        p, va, (((2,), (1,)), ((0,), (0,))),
        preferred_element_type=jnp.float32)
    ob = o.astype(jnp.bfloat16).reshape(heads, rows, dh)
    oc = jnp.concatenate([ob[h] for h in range(heads)], axis=-1)
    o_ref[0] = jax.lax.dot_general(
        oc, w_ref[...], (((1,), (0,)), ((), ())),
        preferred_element_type=jnp.float32)


def kernel(q, k, v, W_o):
    B, H, S, dh = q.shape
    D = H * dh
    rows = NBLK * BLK
    ng = S // rows
    scale = 1.0 / (dh ** 0.5)
    body = functools.partial(_fused_body, heads=H, dh=dh, scale=scale)
    qkv_spec = pl.BlockSpec((1, H, rows, dh), lambda b, n: (b, 0, n, 0))
    out = pl.pallas_call(
        body,
        grid=(B, ng),
        in_specs=[
            qkv_spec,
            qkv_spec,
            qkv_spec,
            pl.BlockSpec((D, D), lambda b, n: (0, 0)),
        ],
        out_specs=pl.BlockSpec((1, rows, D), lambda b, n: (b, n, 0)),
        out_shape=jax.ShapeDtypeStruct((B, S, D), jnp.float32),
        compiler_params=pltpu.CompilerParams(
            dimension_semantics=("parallel", "parallel"),
        ),
    )(q, k, v, W_o.astype(jnp.bfloat16))
    return out


# speedup vs baseline: 3.0086x; 1.0458x over previous
import functools

import jax
import jax.numpy as jnp
from jax.experimental import pallas as pl
from jax.experimental.pallas import tpu as pltpu

BLK = 128
NBLK = 4


def _fused_body(q_ref, k_ref, v_ref, w_ref, o_ref, *, heads, dh, scale):
    rows = NBLK * BLK
    hb = heads * NBLK
    qa = q_ref[0].astype(jnp.bfloat16).reshape(hb, BLK, dh)
    ka = k_ref[0].astype(jnp.bfloat16).reshape(hb, BLK, dh)
    va = v_ref[0].astype(jnp.bfloat16).reshape(hb, BLK, dh)
    s = jax.lax.dot_general(
        qa, ka, (((2,), (2,)), ((0,), (0,))),
        preferred_element_type=jnp.float32) * scale
    m = jnp.max(s, axis=-1, keepdims=True)
    e = jnp.exp(s - m)
    p = (e / jnp.sum(e, axis=-1, keepdims=True)).astype(jnp.bfloat16)
    o = jax.lax.dot_general(
        p, va, (((2,), (1,)), ((0,), (0,))),
        preferred_element_type=jnp.float32)
    ob = o.astype(jnp.bfloat16).reshape(heads, rows, dh)
    oc = jnp.concatenate([ob[h] for h in range(heads)], axis=-1)
    o_ref[0] = jax.lax.dot_general(
        oc, w_ref[...], (((1,), (0,)), ((), ())),
        preferred_element_type=jnp.float32)


def kernel(q, k, v, W_o):
    B, H, S, dh = q.shape
    D = H * dh
    rows = NBLK * BLK
    ng = S // rows
    scale = 1.0 / (dh ** 0.5)
    body = functools.partial(_fused_body, heads=H, dh=dh, scale=scale)
    qkv_spec = pl.BlockSpec((1, H, rows, dh), lambda b, n: (b, 0, n, 0))
    out = pl.pallas_call(
        body,
        grid=(B, ng),
        in_specs=[
            qkv_spec,
            qkv_spec,
            qkv_spec,
            pl.BlockSpec((D, D), lambda b, n: (0, 0)),
        ],
        out_specs=pl.BlockSpec((1, rows, D), lambda b, n: (b, n, 0)),
        out_shape=jax.ShapeDtypeStruct((B, S, D), jnp.float32),
        compiler_params=pltpu.CompilerParams(
            dimension_semantics=("parallel", "parallel"),
        ),
    )(q, k, v, W_o.astype(jnp.bfloat16))
    return out
